# baseline (speedup 1.0000x reference)
import jax
import jax.numpy as jnp
from jax import lax
from jax.experimental import pallas as pl
from jax.experimental.pallas import tpu as pltpu
from jax.experimental.pallas import tpu_sc as plsc

D_MODEL = 2048
D_HIDDEN = 8192
NUM_EXPERTS = 8
TOKENS = 4096

BLK = 576
G = TOKENS // BLK + NUM_EXPERTS
SLOTS = G * BLK
HT = 1024
HK = D_HIDDEN // HT

_LANES = 128
_CHUNK = 512

_NC = 2
_NS = 16
_NW = _NC * _NS
_PER_W = TOKENS // _NW
_SC_CH = 32


def _route_body(x_ref, wg_ref, bg_ref, pos_ref, be_ref, used_ref):
    logits = jnp.dot(x_ref[...], wg_ref[...],
                     preferred_element_type=jnp.float32) + bg_ref[...]
    e = jnp.argmax(logits, axis=1).reshape(TOKENS, 1)
    lane = lax.broadcasted_iota(jnp.int32, (TOKENS, _LANES), 1)
    onehot = (e == lane).astype(jnp.float32)

    tri = (lax.broadcasted_iota(jnp.int32, (_CHUNK, _CHUNK), 0)
           >= lax.broadcasted_iota(jnp.int32, (_CHUNK, _CHUNK), 1)
           ).astype(jnp.float32)
    carry = jnp.zeros((1, _LANES), jnp.float32)
    rank_parts = []
    for c in range(TOKENS // _CHUNK):
        blk = onehot[c * _CHUNK:(c + 1) * _CHUNK, :]
        incl = jnp.dot(tri, blk, preferred_element_type=jnp.float32) + carry
        rank_parts.append(jnp.sum(incl * blk, axis=1, keepdims=True))
        carry = incl[_CHUNK - 1:_CHUNK, :]
    rank_incl = jnp.concatenate(rank_parts, axis=0)
    counts = carry

    blocks = jnp.floor((counts + float(BLK - 1)) * (1.0 / BLK))
    triu = (lax.broadcasted_iota(jnp.int32, (_LANES, _LANES), 0)
            <= lax.broadcasted_iota(jnp.int32, (_LANES, _LANES), 1)
            ).astype(jnp.float32)
    incl_b = jnp.dot(blocks, triu, preferred_element_type=jnp.float32)
    excl_b = incl_b - blocks
    slot_base = excl_b * float(BLK)

    base_tok = jnp.sum(onehot * slot_base, axis=1, keepdims=True)
    pos_ref[...] = (base_tok + rank_incl - 1.0 + 0.5).astype(jnp.int32)

    gi = lax.broadcasted_iota(jnp.int32, (G, _LANES), 0)
    gl = lax.broadcasted_iota(jnp.int32, (G, _LANES), 1)
    excl_i = (excl_b + 0.5).astype(jnp.int32)
    owned = jnp.where((excl_i <= gi) & (gl < NUM_EXPERTS), 1.0, 0.0)
    be_ref[...] = (jnp.sum(owned, axis=1, keepdims=True) - 0.5).astype(jnp.int32)

    lane1 = lax.broadcasted_iota(jnp.int32, (1, _LANES), 1)
    used = jnp.sum(jnp.where(lane1 == NUM_EXPERTS - 1, incl_b, 0.0),
                   axis=1, keepdims=True)
    used_ref[...] = (used + 0.5).astype(jnp.int32)


def _ffn_body(be_ref, used_ref, xs_ref, w1a_ref, w1b_ref, b1_ref, w2_ref,
              b2_ref, out_ref):
    g = pl.program_id(0)
    k = pl.program_id(1)
    DH = D_MODEL // 2

    @pl.when(g < used_ref[0])
    def _():
        e = be_ref[g]
        xb = xs_ref[...].astype(jnp.bfloat16)
        b1s = b1_ref[pl.ds(e * HK + k, 1), :]
        h = (jnp.dot(xb[:, :DH], w1a_ref[0].astype(jnp.bfloat16),
                     preferred_element_type=jnp.float32)
             + jnp.dot(xb[:, DH:], w1b_ref[0].astype(jnp.bfloat16),
                       preferred_element_type=jnp.float32)
             + b1s)
        hb = jnp.maximum(h, 0.0).astype(jnp.bfloat16)
        part = jnp.dot(hb, w2_ref[0].astype(jnp.bfloat16),
                       preferred_element_type=jnp.float32)

        @pl.when(k == 0)
        def _():
            out_ref[...] = part

        @pl.when(k != 0)
        def _():
            out_ref[...] += part

        @pl.when(k == HK - 1)
        def _():
            out_ref[...] += b2_ref[pl.ds(e, 1), :]


def _fzg(g, u):
    return jnp.where(g < u[0], g, u[0] - 1)


def _fzk(g, k, u):
    return jnp.where(g < u[0], k, HK - 1)


def _sc_mesh():
    return plsc.VectorSubcoreMesh(core_axis_name="c", subcore_axis_name="s")


def _scatter_body(x_hbm, pos_hbm, xs_hbm, idx_v, rows_v):
    wid = lax.axis_index("s") * _NC + lax.axis_index("c")
    base = wid * _PER_W

    @pl.loop(0, _PER_W, step=_SC_CH)
    def _(c):
        pltpu.sync_copy(pos_hbm.at[pl.ds(base + c, _SC_CH)], idx_v)
        pltpu.sync_copy(x_hbm.at[pl.ds(base + c, _SC_CH)], rows_v)
        pltpu.sync_copy(rows_v, xs_hbm.at[idx_v])


def _gather_body(ys_hbm, pos_hbm, out_hbm, idx_v, rows_v):
    wid = lax.axis_index("s") * _NC + lax.axis_index("c")
    base = wid * _PER_W

    @pl.loop(0, _PER_W, step=_SC_CH)
    def _(c):
        pltpu.sync_copy(pos_hbm.at[pl.ds(base + c, _SC_CH)], idx_v)
        pltpu.sync_copy(ys_hbm.at[idx_v], rows_v)
        pltpu.sync_copy(rows_v, out_hbm.at[pl.ds(base + c, _SC_CH)])


def kernel(x, Wg, bg, W1, b1, W2, b2):
    wgp = jnp.pad(Wg, ((0, 0), (0, _LANES - NUM_EXPERTS)))
    bgp = jnp.pad(bg, (0, _LANES - NUM_EXPERTS),
                  constant_values=-1e30).reshape(1, _LANES)

    pos2d, be2d, used2d = pl.pallas_call(
        _route_body,
        out_shape=[
            jax.ShapeDtypeStruct((TOKENS, 1), jnp.int32),
            jax.ShapeDtypeStruct((G, 1), jnp.int32),
            jax.ShapeDtypeStruct((1, 1), jnp.int32),
        ],
    )(x, wgp, bgp)
    pos = pos2d.reshape(TOKENS)
    be = be2d.reshape(G)
    used = used2d.reshape(1)

    scatter_k = pl.kernel(
        _scatter_body,
        out_type=jax.ShapeDtypeStruct((SLOTS, D_MODEL), jnp.float32),
        mesh=_sc_mesh(),
        scratch_types=[
            pltpu.VMEM((_SC_CH,), jnp.int32),
            pltpu.VMEM((_SC_CH, D_MODEL), jnp.float32),
        ],
    )
    xs = scatter_k(x, pos)

    grid_spec = pltpu.PrefetchScalarGridSpec(
        num_scalar_prefetch=2,
        grid=(G, HK),
        in_specs=[
            pl.BlockSpec((BLK, D_MODEL), lambda g, k, be, u: (_fzg(g, u), 0)),
            pl.BlockSpec((1, D_MODEL // 2, HT),
                         lambda g, k, be, u: (be[_fzg(g, u)], 0, _fzk(g, k, u))),
            pl.BlockSpec((1, D_MODEL // 2, HT),
                         lambda g, k, be, u: (be[_fzg(g, u)], 1, _fzk(g, k, u))),
            pl.BlockSpec((NUM_EXPERTS * HK, HT), lambda g, k, be, u: (0, 0)),
            pl.BlockSpec((1, HT, D_MODEL),
                         lambda g, k, be, u: (be[_fzg(g, u)], _fzk(g, k, u), 0)),
            pl.BlockSpec((NUM_EXPERTS, D_MODEL), lambda g, k, be, u: (0, 0)),
        ],
        out_specs=pl.BlockSpec((BLK, D_MODEL),
                               lambda g, k, be, u: (_fzg(g, u), 0)),
    )
    ys = pl.pallas_call(
        _ffn_body,
        grid_spec=grid_spec,
        out_shape=jax.ShapeDtypeStruct((SLOTS, D_MODEL), jnp.float32),
        compiler_params=pltpu.CompilerParams(
            dimension_semantics=("parallel", "arbitrary")),
    )(be, used, xs, W1, W1, b1.reshape(NUM_EXPERTS * HK, HT), W2, b2)

    gather_k = pl.kernel(
        _gather_body,
        out_type=jax.ShapeDtypeStruct((TOKENS, D_MODEL), jnp.float32),
        mesh=_sc_mesh(),
        scratch_types=[
            pltpu.VMEM((_SC_CH,), jnp.int32),
            pltpu.VMEM((_SC_CH, D_MODEL), jnp.float32),
        ],
    )
    return gather_k(ys, pos)


# speedup vs baseline: 1.0029x; 1.0029x over previous
import jax
import jax.numpy as jnp
from jax import lax
from jax.experimental import pallas as pl
from jax.experimental.pallas import tpu as pltpu
from jax.experimental.pallas import tpu_sc as plsc

D_MODEL = 2048
D_HIDDEN = 8192
NUM_EXPERTS = 8
TOKENS = 4096

BLK = 576
G = TOKENS // BLK + NUM_EXPERTS
SLOTS = G * BLK
HT = 1024
HK = D_HIDDEN // HT

_LANES = 128
_CHUNK = 512

_NC = 2
_NS = 16
_NW = _NC * _NS
_PER_W = TOKENS // _NW
_SC_CH = 32


def _route_body(x_ref, wg_ref, bg_ref, pos_ref, be_ref, used_ref):
    logits = jnp.dot(x_ref[...], wg_ref[...],
                     preferred_element_type=jnp.float32) + bg_ref[...]
    e = jnp.argmax(logits, axis=1).reshape(TOKENS, 1)
    lane = lax.broadcasted_iota(jnp.int32, (TOKENS, _LANES), 1)
    onehot = (e == lane).astype(jnp.float32)

    tri = (lax.broadcasted_iota(jnp.int32, (_CHUNK, _CHUNK), 0)
           >= lax.broadcasted_iota(jnp.int32, (_CHUNK, _CHUNK), 1)
           ).astype(jnp.float32)
    carry = jnp.zeros((1, _LANES), jnp.float32)
    rank_parts = []
    for c in range(TOKENS // _CHUNK):
        blk = onehot[c * _CHUNK:(c + 1) * _CHUNK, :]
        incl = jnp.dot(tri, blk, preferred_element_type=jnp.float32) + carry
        rank_parts.append(jnp.sum(incl * blk, axis=1, keepdims=True))
        carry = incl[_CHUNK - 1:_CHUNK, :]
    rank_incl = jnp.concatenate(rank_parts, axis=0)
    counts = carry

    blocks = jnp.floor((counts + float(BLK - 1)) * (1.0 / BLK))
    triu = (lax.broadcasted_iota(jnp.int32, (_LANES, _LANES), 0)
            <= lax.broadcasted_iota(jnp.int32, (_LANES, _LANES), 1)
            ).astype(jnp.float32)
    incl_b = jnp.dot(blocks, triu, preferred_element_type=jnp.float32)
    excl_b = incl_b - blocks
    slot_base = excl_b * float(BLK)

    base_tok = jnp.sum(onehot * slot_base, axis=1, keepdims=True)
    pos_ref[...] = (base_tok + rank_incl - 1.0 + 0.5).astype(jnp.int32)

    gi = lax.broadcasted_iota(jnp.int32, (G, _LANES), 0)
    gl = lax.broadcasted_iota(jnp.int32, (G, _LANES), 1)
    excl_i = (excl_b + 0.5).astype(jnp.int32)
    owned = jnp.where((excl_i <= gi) & (gl < NUM_EXPERTS), 1.0, 0.0)
    be_ref[...] = (jnp.sum(owned, axis=1, keepdims=True) - 0.5).astype(jnp.int32)

    lane1 = lax.broadcasted_iota(jnp.int32, (1, _LANES), 1)
    used = jnp.sum(jnp.where(lane1 == NUM_EXPERTS - 1, incl_b, 0.0),
                   axis=1, keepdims=True)
    used_ref[...] = (used + 0.5).astype(jnp.int32)


def _ffn_body(be_ref, used_ref, xs_ref, w1_ref, b1_ref, w2_ref, b2_ref,
              out_ref):
    g = pl.program_id(0)
    k = pl.program_id(1)

    @pl.when(g < used_ref[0])
    def _():
        e = be_ref[g]
        xb = xs_ref[...].astype(jnp.bfloat16)
        b1s = b1_ref[pl.ds(e * HK + k, 1), :]
        h = jnp.dot(xb, w1_ref[0].astype(jnp.bfloat16),
                    preferred_element_type=jnp.float32) + b1s
        hb = jnp.maximum(h, 0.0).astype(jnp.bfloat16)
        part = jnp.dot(hb, w2_ref[0].astype(jnp.bfloat16),
                       preferred_element_type=jnp.float32)

        @pl.when(k == 0)
        def _():
            out_ref[...] = part

        @pl.when(k != 0)
        def _():
            out_ref[...] += part

        @pl.when(k == HK - 1)
        def _():
            out_ref[...] += b2_ref[pl.ds(e, 1), :]


def _fzg(g, u):
    return jnp.where(g < u[0], g, u[0] - 1)


def _fzk(g, k, u):
    return jnp.where(g < u[0], k, HK - 1)


def _sc_mesh():
    return plsc.VectorSubcoreMesh(core_axis_name="c", subcore_axis_name="s")


def _scatter_body(x_hbm, pos_hbm, xs_hbm, idx_v, rows_v):
    wid = lax.axis_index("s") * _NC + lax.axis_index("c")
    base = wid * _PER_W

    @pl.loop(0, _PER_W, step=_SC_CH)
    def _(c):
        pltpu.sync_copy(pos_hbm.at[pl.ds(base + c, _SC_CH)], idx_v)
        pltpu.sync_copy(x_hbm.at[pl.ds(base + c, _SC_CH)], rows_v)
        pltpu.sync_copy(rows_v, xs_hbm.at[idx_v])


def _gather_body(ys_hbm, pos_hbm, out_hbm, idx_v, rows_v):
    wid = lax.axis_index("s") * _NC + lax.axis_index("c")
    base = wid * _PER_W

    @pl.loop(0, _PER_W, step=_SC_CH)
    def _(c):
        pltpu.sync_copy(pos_hbm.at[pl.ds(base + c, _SC_CH)], idx_v)
        pltpu.sync_copy(ys_hbm.at[idx_v], rows_v)
        pltpu.sync_copy(rows_v, out_hbm.at[pl.ds(base + c, _SC_CH)])


def kernel(x, Wg, bg, W1, b1, W2, b2):
    wgp = jnp.pad(Wg, ((0, 0), (0, _LANES - NUM_EXPERTS)))
    bgp = jnp.pad(bg, (0, _LANES - NUM_EXPERTS),
                  constant_values=-1e30).reshape(1, _LANES)

    pos2d, be2d, used2d = pl.pallas_call(
        _route_body,
        out_shape=[
            jax.ShapeDtypeStruct((TOKENS, 1), jnp.int32),
            jax.ShapeDtypeStruct((G, 1), jnp.int32),
            jax.ShapeDtypeStruct((1, 1), jnp.int32),
        ],
    )(x, wgp, bgp)
    pos = pos2d.reshape(TOKENS)
    be = be2d.reshape(G)
    used = used2d.reshape(1)

    scatter_k = pl.kernel(
        _scatter_body,
        out_type=jax.ShapeDtypeStruct((SLOTS, D_MODEL), jnp.float32),
        mesh=_sc_mesh(),
        scratch_types=[
            pltpu.VMEM((_SC_CH,), jnp.int32),
            pltpu.VMEM((_SC_CH, D_MODEL), jnp.float32),
        ],
    )
    xs = scatter_k(x, pos)

    grid_spec = pltpu.PrefetchScalarGridSpec(
        num_scalar_prefetch=2,
        grid=(G, HK),
        in_specs=[
            pl.BlockSpec((BLK, D_MODEL), lambda g, k, be, u: (_fzg(g, u), 0)),
            pl.BlockSpec((1, D_MODEL, HT),
                         lambda g, k, be, u: (be[_fzg(g, u)], 0, _fzk(g, k, u))),
            pl.BlockSpec((NUM_EXPERTS * HK, HT), lambda g, k, be, u: (0, 0)),
            pl.BlockSpec((1, HT, D_MODEL),
                         lambda g, k, be, u: (be[_fzg(g, u)], _fzk(g, k, u), 0)),
            pl.BlockSpec((NUM_EXPERTS, D_MODEL), lambda g, k, be, u: (0, 0)),
        ],
        out_specs=pl.BlockSpec((BLK, D_MODEL),
                               lambda g, k, be, u: (_fzg(g, u), 0)),
    )
    ys = pl.pallas_call(
        _ffn_body,
        grid_spec=grid_spec,
        out_shape=jax.ShapeDtypeStruct((SLOTS, D_MODEL), jnp.float32),
        compiler_params=pltpu.CompilerParams(
            dimension_semantics=("parallel", "arbitrary")),
    )(be, used, xs, W1, b1.reshape(NUM_EXPERTS * HK, HT), W2, b2)

    gather_k = pl.kernel(
        _gather_body,
        out_type=jax.ShapeDtypeStruct((TOKENS, D_MODEL), jnp.float32),
        mesh=_sc_mesh(),
        scratch_types=[
            pltpu.VMEM((_SC_CH,), jnp.int32),
            pltpu.VMEM((_SC_CH, D_MODEL), jnp.float32),
        ],
    )
    return gather_k(ys, pos)


# speedup vs baseline: 1.0352x; 1.0322x over previous
import jax
import jax.numpy as jnp
from jax import lax
from jax.experimental import pallas as pl
from jax.experimental.pallas import tpu as pltpu
from jax.experimental.pallas import tpu_sc as plsc

D_MODEL = 2048
D_HIDDEN = 8192
NUM_EXPERTS = 8
TOKENS = 4096

BLK = 640
G = TOKENS // BLK + NUM_EXPERTS
SLOTS = G * BLK
HT = 1024
HK = D_HIDDEN // HT

_LANES = 128
_CHUNK = 512

_NC = 2
_NS = 16
_NW = _NC * _NS
_PER_W = TOKENS // _NW
_SC_CH = 32


def _route_body(x_ref, wg_ref, bg_ref, pos_ref, be_ref, used_ref):
    logits = jnp.dot(x_ref[...], wg_ref[...],
                     preferred_element_type=jnp.float32) + bg_ref[...]
    e = jnp.argmax(logits, axis=1).reshape(TOKENS, 1)
    lane = lax.broadcasted_iota(jnp.int32, (TOKENS, _LANES), 1)
    onehot = (e == lane).astype(jnp.float32)

    tri = (lax.broadcasted_iota(jnp.int32, (_CHUNK, _CHUNK), 0)
           >= lax.broadcasted_iota(jnp.int32, (_CHUNK, _CHUNK), 1)
           ).astype(jnp.float32)
    carry = jnp.zeros((1, _LANES), jnp.float32)
    rank_parts = []
    for c in range(TOKENS // _CHUNK):
        blk = onehot[c * _CHUNK:(c + 1) * _CHUNK, :]
        incl = jnp.dot(tri, blk, preferred_element_type=jnp.float32) + carry
        rank_parts.append(jnp.sum(incl * blk, axis=1, keepdims=True))
        carry = incl[_CHUNK - 1:_CHUNK, :]
    rank_incl = jnp.concatenate(rank_parts, axis=0)
    counts = carry

    blocks = jnp.floor((counts + float(BLK - 1)) * (1.0 / BLK))
    triu = (lax.broadcasted_iota(jnp.int32, (_LANES, _LANES), 0)
            <= lax.broadcasted_iota(jnp.int32, (_LANES, _LANES), 1)
            ).astype(jnp.float32)
    incl_b = jnp.dot(blocks, triu, preferred_element_type=jnp.float32)
    excl_b = incl_b - blocks
    slot_base = excl_b * float(BLK)

    base_tok = jnp.sum(onehot * slot_base, axis=1, keepdims=True)
    pos_ref[...] = (base_tok + rank_incl - 1.0 + 0.5).astype(jnp.int32)

    gi = lax.broadcasted_iota(jnp.int32, (G, _LANES), 0)
    gl = lax.broadcasted_iota(jnp.int32, (G, _LANES), 1)
    excl_i = (excl_b + 0.5).astype(jnp.int32)
    owned = jnp.where((excl_i <= gi) & (gl < NUM_EXPERTS), 1.0, 0.0)
    be_ref[...] = (jnp.sum(owned, axis=1, keepdims=True) - 0.5).astype(jnp.int32)

    lane1 = lax.broadcasted_iota(jnp.int32, (1, _LANES), 1)
    used = jnp.sum(jnp.where(lane1 == NUM_EXPERTS - 1, incl_b, 0.0),
                   axis=1, keepdims=True)
    used_ref[...] = (used + 0.5).astype(jnp.int32)


def _ffn_body(be_ref, used_ref, xs_ref, w1_ref, b1_ref, w2_ref, b2_ref,
              out_ref):
    g = pl.program_id(0)
    k = pl.program_id(1)

    @pl.when(g < used_ref[0])
    def _():
        e = be_ref[g]
        xb = xs_ref[...].astype(jnp.bfloat16)
        b1s = b1_ref[pl.ds(e * HK + k, 1), :]
        h = jnp.dot(xb, w1_ref[0].astype(jnp.bfloat16),
                    preferred_element_type=jnp.float32) + b1s
        hb = jnp.maximum(h, 0.0).astype(jnp.bfloat16)
        part = jnp.dot(hb, w2_ref[0].astype(jnp.bfloat16),
                       preferred_element_type=jnp.float32)

        @pl.when(k == 0)
        def _():
            out_ref[...] = part

        @pl.when(k != 0)
        def _():
            out_ref[...] += part

        @pl.when(k == HK - 1)
        def _():
            out_ref[...] += b2_ref[pl.ds(e, 1), :]


def _fzg(g, u):
    return jnp.where(g < u[0], g, u[0] - 1)


def _fzk(g, k, u):
    return jnp.where(g < u[0], k, HK - 1)


def _sc_mesh():
    return plsc.VectorSubcoreMesh(core_axis_name="c", subcore_axis_name="s")


def _scatter_body(x_hbm, pos_hbm, xs_hbm, idx_v, rows_v):
    wid = lax.axis_index("s") * _NC + lax.axis_index("c")
    base = wid * _PER_W

    @pl.loop(0, _PER_W, step=_SC_CH)
    def _(c):
        pltpu.sync_copy(pos_hbm.at[pl.ds(base + c, _SC_CH)], idx_v)
        pltpu.sync_copy(x_hbm.at[pl.ds(base + c, _SC_CH)], rows_v)
        pltpu.sync_copy(rows_v, xs_hbm.at[idx_v])


def _gather_body(ys_hbm, pos_hbm, out_hbm, idx_v, rows_v):
    wid = lax.axis_index("s") * _NC + lax.axis_index("c")
    base = wid * _PER_W

    @pl.loop(0, _PER_W, step=_SC_CH)
    def _(c):
        pltpu.sync_copy(pos_hbm.at[pl.ds(base + c, _SC_CH)], idx_v)
        pltpu.sync_copy(ys_hbm.at[idx_v], rows_v)
        pltpu.sync_copy(rows_v, out_hbm.at[pl.ds(base + c, _SC_CH)])


def kernel(x, Wg, bg, W1, b1, W2, b2):
    wgp = jnp.pad(Wg, ((0, 0), (0, _LANES - NUM_EXPERTS)))
    bgp = jnp.pad(bg, (0, _LANES - NUM_EXPERTS),
                  constant_values=-1e30).reshape(1, _LANES)

    pos2d, be2d, used2d = pl.pallas_call(
        _route_body,
        out_shape=[
            jax.ShapeDtypeStruct((TOKENS, 1), jnp.int32),
            jax.ShapeDtypeStruct((G, 1), jnp.int32),
            jax.ShapeDtypeStruct((1, 1), jnp.int32),
        ],
    )(x, wgp, bgp)
    pos = pos2d.reshape(TOKENS)
    be = be2d.reshape(G)
    used = used2d.reshape(1)

    scatter_k = pl.kernel(
        _scatter_body,
        out_type=jax.ShapeDtypeStruct((SLOTS, D_MODEL), jnp.float32),
        mesh=_sc_mesh(),
        scratch_types=[
            pltpu.VMEM((_SC_CH,), jnp.int32),
            pltpu.VMEM((_SC_CH, D_MODEL), jnp.float32),
        ],
    )
    xs = scatter_k(x, pos)

    grid_spec = pltpu.PrefetchScalarGridSpec(
        num_scalar_prefetch=2,
        grid=(G, HK),
        in_specs=[
            pl.BlockSpec((BLK, D_MODEL), lambda g, k, be, u: (_fzg(g, u), 0)),
            pl.BlockSpec((1, D_MODEL, HT),
                         lambda g, k, be, u: (be[_fzg(g, u)], 0, _fzk(g, k, u))),
            pl.BlockSpec((NUM_EXPERTS * HK, HT), lambda g, k, be, u: (0, 0)),
            pl.BlockSpec((1, HT, D_MODEL),
                         lambda g, k, be, u: (be[_fzg(g, u)], _fzk(g, k, u), 0)),
            pl.BlockSpec((NUM_EXPERTS, D_MODEL), lambda g, k, be, u: (0, 0)),
        ],
        out_specs=pl.BlockSpec((BLK, D_MODEL),
                               lambda g, k, be, u: (_fzg(g, u), 0)),
    )
    ys = pl.pallas_call(
        _ffn_body,
        grid_spec=grid_spec,
        out_shape=jax.ShapeDtypeStruct((SLOTS, D_MODEL), jnp.float32),
        compiler_params=pltpu.CompilerParams(
            dimension_semantics=("parallel", "arbitrary")),
    )(be, used, xs, W1, b1.reshape(NUM_EXPERTS * HK, HT), W2, b2)

    gather_k = pl.kernel(
        _gather_body,
        out_type=jax.ShapeDtypeStruct((TOKENS, D_MODEL), jnp.float32),
        mesh=_sc_mesh(),
        scratch_types=[
            pltpu.VMEM((_SC_CH,), jnp.int32),
            pltpu.VMEM((_SC_CH, D_MODEL), jnp.float32),
        ],
    )
    return gather_k(ys, pos)
